# Initial kernel scaffold; baseline (speedup 1.0000x reference)
import jax
import jax.numpy as jnp
from jax.experimental import pallas as pl


def kernel(x, edge_index, batch, W1, b1, W2, b2, W3, b3, g01, be01, g02, be02, g03, be03, Wg1a, bg1a, Wg1b, bg1b, g21, be21, Wg2a, bg2a, Wg2b, bg2b, g22, be22, Wg3a, bg3a, Wg3b, bg3b, g23, be23, Wf1, bf1, Wf2, bf2, Wf3, bf3):
    raise NotImplementedError("write your pallas kernel here")


# speedup vs baseline: 2.2702x; 2.2702x over previous
import functools

import jax
import jax.numpy as jnp
from jax import lax
from jax.experimental import pallas as pl
from jax.experimental.pallas import tpu as pltpu
from jax.experimental.pallas import tpu_sc as plsc

N = 10000
G = 64
NC = 2
NS = 16
NW = NC * NS
BN_ROWS = 400
GRID = N // BN_ROWS
NP = 10240
GP = 128
EPS = 1e-5


def _make_scatter(D, n_out, e_pad, k, gather=True):
    epw = e_pad // NW
    nchunks = epw // k
    rows_per_sub = n_out // NS
    mesh = plsc.VectorSubcoreMesh(core_axis_name="c", subcore_axis_name="s")

    @functools.partial(
        pl.kernel,
        mesh=mesh,
        out_type=jax.ShapeDtypeStruct((NC, n_out, D), jnp.float32),
        scratch_types=[
            pltpu.VMEM((k,), jnp.int32),
            pltpu.VMEM((k,), jnp.int32),
            pltpu.VMEM((k, D), jnp.float32),
            pltpu.VMEM_SHARED((n_out, D), jnp.float32),
            pltpu.SemaphoreType.DMA,
        ],
    )
    def sck(feat_hbm, src_hbm, dst_hbm, zeros_hbm, out_hbm, sidx, didx, rows,
            acc, sem):
        cid = lax.axis_index("c")
        sid = lax.axis_index("s")
        wid = sid * NC + cid
        rbase = sid * rows_per_sub
        pltpu.sync_copy(zeros_hbm.at[pl.ds(rbase, rows_per_sub)],
                        acc.at[pl.ds(rbase, rows_per_sub)])
        plsc.subcore_barrier()
        if not gather:
            pltpu.sync_copy(feat_hbm, rows)
        ebase = wid * epw

        def body(j, carry):
            off = ebase + j * k
            pltpu.sync_copy(dst_hbm.at[pl.ds(off, k)], didx)
            if gather:
                pltpu.sync_copy(src_hbm.at[pl.ds(off, k)], sidx)
                pltpu.async_copy(feat_hbm.at[sidx], rows, sem).wait()
            pltpu.sync_copy(rows, acc.at[didx], add=True)
            return carry

        lax.fori_loop(0, nchunks, body, 0)
        plsc.subcore_barrier()
        pltpu.sync_copy(acc.at[pl.ds(rbase, rows_per_sub)],
                        out_hbm.at[cid, pl.ds(rbase, rows_per_sub)])

    return sck


def _deg(degp):
    return degp[0, :, 15:16] + degp[1, :, 15:16] + 1.0


def _dinv(degp):
    return lax.rsqrt(jnp.maximum(_deg(degp), 1e-12))


def _bn_coefs(stats, g, be):
    mu = stats[0:1, :] / N
    var = stats[1:2, :] / N - mu * mu
    a = g * lax.rsqrt(var + EPS)
    return a, be - mu * a


def _row_specs(dims):
    return [pl.BlockSpec((BN_ROWS, d), lambda i: (i, 0)) for d in dims]


def _part_spec(d):
    return pl.BlockSpec((NC, BN_ROWS, d), lambda i: (0, i, 0))


def _full(shape):
    return pl.BlockSpec(shape, lambda i: tuple(0 for _ in shape))


def _stats_update(stats_ref, y, i):
    @pl.when(i == 0)
    def _():
        stats_ref[...] = jnp.zeros_like(stats_ref)

    stats_ref[0:1, :] += jnp.sum(y, axis=0, keepdims=True)
    stats_ref[1:2, :] += jnp.sum(y * y, axis=0, keepdims=True)


def _mm_dinv(x, W, degp, stats=None, g=None, be=None):
    K, D = W.shape

    def body(*refs):
        if stats is None:
            x_ref, w_ref, d_ref, o_ref = refs
            xv = x_ref[...]
        else:
            x_ref, w_ref, d_ref, s_ref, g_ref, b_ref, o_ref = refs
            a, c = _bn_coefs(s_ref[...], g_ref[...], b_ref[...])
            xv = x_ref[...] * a + c
        o_ref[...] = jnp.dot(xv, w_ref[...],
                             preferred_element_type=jnp.float32) * _dinv(d_ref[...])

    ins = [x, W, degp]
    specs = _row_specs([K]) + [_full((K, D)), _part_spec(16)]
    if stats is not None:
        ins += [stats, g, be]
        specs += [_full((8, K)), _full((1, K)), _full((1, K))]
    return pl.pallas_call(
        body,
        grid=(GRID,),
        in_specs=specs,
        out_specs=pl.BlockSpec((BN_ROWS, D), lambda i: (i, 0)),
        out_shape=jax.ShapeDtypeStruct((N, D), jnp.float32),
    )(*ins)


def _gcn_fin(hs, parts, degp, b):
    D = hs.shape[1]

    def body(*refs):
        h_ref = refs[0]
        p_refs = refs[1:1 + len(parts)]
        d_ref, b_ref, o_ref, s_ref = refs[1 + len(parts):]
        agg = jnp.concatenate([p[0] + p[1] for p in p_refs], axis=-1)
        y = jnp.maximum(_dinv(d_ref[...]) * (agg + h_ref[...]) + b_ref[...], 0.0)
        o_ref[...] = y
        _stats_update(s_ref, y, pl.program_id(0))

    specs = (_row_specs([D]) + [_part_spec(p.shape[2]) for p in parts]
             + [_part_spec(16), _full((1, D))])
    return pl.pallas_call(
        body,
        grid=(GRID,),
        in_specs=specs,
        out_specs=[pl.BlockSpec((BN_ROWS, D), lambda i: (i, 0)),
                   _full((8, D))],
        out_shape=[jax.ShapeDtypeStruct((N, D), jnp.float32),
                   jax.ShapeDtypeStruct((8, D), jnp.float32)],
    )(hs, *parts, degp, b)


def _gin_a(z, parts, Wa, ba, degp=None, stats=None, g=None, be=None):
    K, D = Wa.shape

    def body(*refs):
        z_ref = refs[0]
        p_refs = refs[1:1 + len(parts)]
        rest = list(refs[1 + len(parts):])
        agg = jnp.concatenate([p[0] + p[1] for p in p_refs], axis=-1)
        if stats is None:
            w_ref, b_ref, o_ref = rest
            h = z_ref[...] + agg
        else:
            d_ref, s_ref, g_ref, be_ref, w_ref, b_ref, o_ref = rest
            a, c = _bn_coefs(s_ref[...], g_ref[...], be_ref[...])
            h = a * (z_ref[...] + agg) + c * _deg(d_ref[...])
        o_ref[...] = jnp.maximum(
            jnp.dot(h, w_ref[...], preferred_element_type=jnp.float32)
            + b_ref[...], 0.0)

    ins = [z] + list(parts)
    specs = _row_specs([K]) + [_part_spec(p.shape[2]) for p in parts]
    if stats is not None:
        ins += [degp, stats, g, be]
        specs += [_part_spec(16), _full((8, K)), _full((1, K)), _full((1, K))]
    ins += [Wa, ba]
    specs += [_full((K, D)), _full((1, D))]
    return pl.pallas_call(
        body,
        grid=(GRID,),
        in_specs=specs,
        out_specs=pl.BlockSpec((BN_ROWS, D), lambda i: (i, 0)),
        out_shape=jax.ShapeDtypeStruct((N, D), jnp.float32),
    )(*ins)


def _mm_relu_stats(u, W, b):
    K, D = W.shape

    def body(u_ref, w_ref, b_ref, o_ref, s_ref):
        y = jnp.maximum(
            jnp.dot(u_ref[...], w_ref[...], preferred_element_type=jnp.float32)
            + b_ref[...], 0.0)
        o_ref[...] = y
        _stats_update(s_ref, y, pl.program_id(0))

    return pl.pallas_call(
        body,
        grid=(GRID,),
        in_specs=_row_specs([K]) + [_full((K, D)), _full((1, D))],
        out_specs=[pl.BlockSpec((BN_ROWS, D), lambda i: (i, 0)),
                   _full((8, D))],
        out_shape=[jax.ShapeDtypeStruct((N, D), jnp.float32),
                   jax.ShapeDtypeStruct((8, D), jnp.float32)],
    )(u, W, b)


def _final(ph, pz, stats_h, g03, be03, stats_z, g23, be23,
           Wf1, bf1, Wf2, bf2, Wf3, bf3):
    def body(ph_ref, pz_ref, sh_ref, gh_ref, bh_ref, sz_ref, gz_ref,
             bz_ref, w1_ref, b1_ref, w2_ref, b2_ref, w3_ref, b3_ref, o_ref):
        cnt = pz_ref[0, 0:G, 127:128] + pz_ref[1, 0:G, 127:128]
        ah, ch = _bn_coefs(sh_ref[...], gh_ref[...], bh_ref[...])
        az, cz = _bn_coefs(sz_ref[...], gz_ref[...], bz_ref[...])
        hp = ah * (ph_ref[0, 0:G, :] + ph_ref[1, 0:G, :]) + ch * cnt
        zp = az * (pz_ref[0, 0:G, :] + pz_ref[1, 0:G, :]) + cz * cnt
        cr = jnp.concatenate([hp, zp[:, 0:64]], axis=-1)
        cr = jnp.maximum(
            jnp.dot(cr, w1_ref[...], preferred_element_type=jnp.float32)
            + b1_ref[...], 0.0)
        cr = jnp.maximum(
            jnp.dot(cr, w2_ref[...], preferred_element_type=jnp.float32)
            + b2_ref[...], 0.0)
        cr = jnp.dot(cr, w3_ref[...], preferred_element_type=jnp.float32) \
            + b3_ref[...]
        o_ref[...] = jnp.maximum(cr, 0.0)

    return pl.pallas_call(
        body,
        out_shape=jax.ShapeDtypeStruct((G, 1), jnp.float32),
    )(ph, pz, stats_h, g03, be03, stats_z, g23, be23,
      Wf1, bf1, Wf2, bf2, Wf3, bf3)


def kernel(x, edge_index, batch, W1, b1, W2, b2, W3, b3, g01, be01, g02, be02,
           g03, be03, Wg1a, bg1a, Wg1b, bg1b, g21, be21, Wg2a, bg2a, Wg2b,
           bg2b, g22, be22, Wg3a, bg3a, Wg3b, bg3b, g23, be23, Wf1, bf1, Wf2,
           bf2, Wf3, bf3):
    E = edge_index.shape[1]
    E_PAD = 163840
    src = edge_index[0]
    dst = edge_index[1]
    src_p = jnp.concatenate(
        [src, jnp.zeros((E_PAD - E,), jnp.int32)])
    dst_p = jnp.concatenate(
        [dst, jnp.full((E_PAD - E,), N, jnp.int32)])
    NSRC_PAD = 10240
    psrc = jnp.concatenate(
        [jnp.arange(N, dtype=jnp.int32),
         jnp.zeros((NSRC_PAD - N,), jnp.int32)])
    pdst = jnp.concatenate(
        [batch.astype(jnp.int32), jnp.full((NSRC_PAD - N,), G, jnp.int32)])

    zNP128 = jnp.zeros((NP, 128), jnp.float32)
    zGP128 = jnp.zeros((GP, 128), jnp.float32)

    sc_edge128 = _make_scatter(128, NP, E_PAD, 128)
    sc_pool128 = _make_scatter(128, GP, NSRC_PAD, 80)

    def edge_scatter(feat):
        return [sc_edge128(feat[:, j:j + 128], src_p, dst_p, zNP128)
                for j in range(0, feat.shape[1], 128)]

    xp = jnp.pad(x, ((0, 0), (0, 11)))
    xp = xp.at[:, 383].set(1.0)
    W1p = jnp.pad(W1, ((0, 11), (0, 0)))
    Wg1ap = jnp.pad(Wg1a, ((0, 11), (0, 0)))
    Wg3bp = jnp.pad(Wg3b, ((0, 0), (0, 64)))
    bg3bp = jnp.pad(bg3b, (0, 64)).at[127].set(1.0)
    g23p = jnp.pad(g23, (0, 64))
    be23p = jnp.pad(be23, (0, 64))
    r1 = lambda v: v.reshape(1, -1)

    px = edge_scatter(xp)
    degp = lax.slice(px[2], (0, 0, 112), (NC, NP, 128))

    hs1 = _mm_dinv(xp, W1p, degp)
    y1, st1 = _gcn_fin(hs1, edge_scatter(hs1), degp, r1(b1))
    hs2 = _mm_dinv(y1, W2, degp, st1, r1(g01), r1(be01))
    y2, st2 = _gcn_fin(hs2, edge_scatter(hs2), degp, r1(b2))
    hs3 = _mm_dinv(y2, W3, degp, st2, r1(g02), r1(be02))
    y3, st3 = _gcn_fin(hs3, edge_scatter(hs3), degp, r1(b3))
    ph = sc_pool128(y3, psrc, pdst, zGP128)

    u1 = _gin_a(xp, px, Wg1ap, r1(bg1a))
    z1, sz1 = _mm_relu_stats(u1, Wg1b, r1(bg1b))
    u2 = _gin_a(z1, edge_scatter(z1), Wg2a, r1(bg2a),
                degp, sz1, r1(g21), r1(be21))
    z2, sz2 = _mm_relu_stats(u2, Wg2b, r1(bg2b))
    u3 = _gin_a(z2, edge_scatter(z2), Wg3a, r1(bg3a),
                degp, sz2, r1(g22), r1(be22))
    z3, sz3 = _mm_relu_stats(u3, Wg3bp, r1(bg3bp))
    pz = sc_pool128(z3, psrc, pdst, zGP128)

    out = _final(ph, pz, st3, r1(g03), r1(be03), sz3, r1(g23p),
                 r1(be23p), Wf1, r1(bf1), Wf2, r1(bf2), Wf3, r1(bf3))
    return out.reshape(-1)


# speedup vs baseline: 2.7506x; 1.2116x over previous
import functools

import jax
import jax.numpy as jnp
from jax import lax
from jax.experimental import pallas as pl
from jax.experimental.pallas import tpu as pltpu
from jax.experimental.pallas import tpu_sc as plsc

N = 10000
G = 64
NC = 2
NS = 16
NW = NC * NS
BN_ROWS = 400
GRID = N // BN_ROWS
NP = 10240
GP = 128
EPS = 1e-5


def _make_scatter(D, n_out, e_pad, k):
    epw = e_pad // NW
    nchunks = epw // k
    NBUF = 2
    ngroups = nchunks // NBUF
    rows_per_sub = n_out // NS
    mesh = plsc.VectorSubcoreMesh(core_axis_name="c", subcore_axis_name="s")

    @functools.partial(
        pl.kernel,
        mesh=mesh,
        out_type=jax.ShapeDtypeStruct((NC, n_out, D), jnp.float32),
        scratch_types=[
            pltpu.VMEM((nchunks, k), jnp.int32),
            pltpu.VMEM((nchunks, k), jnp.int32),
            pltpu.VMEM((NBUF, k, D), jnp.float32),
            pltpu.VMEM_SHARED((n_out, D), jnp.float32),
        ] + [pltpu.SemaphoreType.DMA] * NBUF,
    )
    def sck(feat_hbm, src_hbm, dst_hbm, zeros_hbm, out_hbm, sidx, didx, rows,
            acc, *sems):
        cid = lax.axis_index("c")
        sid = lax.axis_index("s")
        wid = sid * NC + cid
        rbase = sid * rows_per_sub
        pltpu.sync_copy(src_hbm.at[wid], sidx)
        pltpu.sync_copy(dst_hbm.at[wid], didx)
        pltpu.sync_copy(zeros_hbm.at[pl.ds(rbase, rows_per_sub)],
                        acc.at[pl.ds(rbase, rows_per_sub)])
        plsc.subcore_barrier()
        for b in range(NBUF):
            pltpu.async_copy(feat_hbm.at[sidx.at[b]], rows.at[b], sems[b])

        def gbody(g, carry):
            for b in range(NBUF):
                j = g * NBUF + b
                pltpu.make_async_copy(feat_hbm.at[sidx.at[b]], rows.at[b],
                                      sems[b]).wait()
                pltpu.sync_copy(rows.at[b], acc.at[didx.at[j]], add=True)

                @pl.when(j + NBUF < nchunks)
                def _():
                    pltpu.async_copy(feat_hbm.at[sidx.at[j + NBUF]],
                                     rows.at[b], sems[b])
            return carry

        lax.fori_loop(0, ngroups, gbody, 0)
        plsc.subcore_barrier()
        pltpu.sync_copy(acc.at[pl.ds(rbase, rows_per_sub)],
                        out_hbm.at[cid, pl.ds(rbase, rows_per_sub)])

    return sck


def _deg(degp):
    return degp[0, :, 15:16] + degp[1, :, 15:16] + 1.0


def _dinv(degp):
    return lax.rsqrt(jnp.maximum(_deg(degp), 1e-12))


def _bn_coefs(stats, g, be):
    mu = stats[0:1, :] / N
    var = stats[1:2, :] / N - mu * mu
    a = g * lax.rsqrt(var + EPS)
    return a, be - mu * a


def _row_specs(dims):
    return [pl.BlockSpec((BN_ROWS, d), lambda i: (i, 0)) for d in dims]


def _part_spec(d):
    return pl.BlockSpec((NC, BN_ROWS, d), lambda i: (0, i, 0))


def _full(shape):
    return pl.BlockSpec(shape, lambda i: tuple(0 for _ in shape))


def _stats_update(stats_ref, y, i):
    @pl.when(i == 0)
    def _():
        stats_ref[...] = jnp.zeros_like(stats_ref)

    stats_ref[0:1, :] += jnp.sum(y, axis=0, keepdims=True)
    stats_ref[1:2, :] += jnp.sum(y * y, axis=0, keepdims=True)


def _mm_dinv(x, W, degp, stats=None, g=None, be=None):
    K, D = W.shape

    def body(*refs):
        if stats is None:
            x_ref, w_ref, d_ref, o_ref = refs
            xv = x_ref[...]
        else:
            x_ref, w_ref, d_ref, s_ref, g_ref, b_ref, o_ref = refs
            a, c = _bn_coefs(s_ref[...], g_ref[...], b_ref[...])
            xv = x_ref[...] * a + c
        o_ref[...] = jnp.dot(xv, w_ref[...],
                             preferred_element_type=jnp.float32) * _dinv(d_ref[...])

    ins = [x, W, degp]
    specs = _row_specs([K]) + [_full((K, D)), _part_spec(16)]
    if stats is not None:
        ins += [stats, g, be]
        specs += [_full((8, K)), _full((1, K)), _full((1, K))]
    return pl.pallas_call(
        body,
        grid=(GRID,),
        in_specs=specs,
        out_specs=pl.BlockSpec((BN_ROWS, D), lambda i: (i, 0)),
        out_shape=jax.ShapeDtypeStruct((N, D), jnp.float32),
    )(*ins)


def _gcn_fin(hs, parts, degp, b):
    D = hs.shape[1]

    def body(*refs):
        h_ref = refs[0]
        p_refs = refs[1:1 + len(parts)]
        d_ref, b_ref, o_ref, s_ref = refs[1 + len(parts):]
        agg = jnp.concatenate([p[0] + p[1] for p in p_refs], axis=-1)
        y = jnp.maximum(_dinv(d_ref[...]) * (agg + h_ref[...]) + b_ref[...], 0.0)
        o_ref[...] = y
        _stats_update(s_ref, y, pl.program_id(0))

    specs = (_row_specs([D]) + [_part_spec(p.shape[2]) for p in parts]
             + [_part_spec(16), _full((1, D))])
    return pl.pallas_call(
        body,
        grid=(GRID,),
        in_specs=specs,
        out_specs=[pl.BlockSpec((BN_ROWS, D), lambda i: (i, 0)),
                   _full((8, D))],
        out_shape=[jax.ShapeDtypeStruct((N, D), jnp.float32),
                   jax.ShapeDtypeStruct((8, D), jnp.float32)],
    )(hs, *parts, degp, b)


def _gin_a(z, parts, Wa, ba, degp=None, stats=None, g=None, be=None):
    K, D = Wa.shape

    def body(*refs):
        z_ref = refs[0]
        p_refs = refs[1:1 + len(parts)]
        rest = list(refs[1 + len(parts):])
        agg = jnp.concatenate([p[0] + p[1] for p in p_refs], axis=-1)
        if stats is None:
            w_ref, b_ref, o_ref = rest
            h = z_ref[...] + agg
        else:
            d_ref, s_ref, g_ref, be_ref, w_ref, b_ref, o_ref = rest
            a, c = _bn_coefs(s_ref[...], g_ref[...], be_ref[...])
            h = a * (z_ref[...] + agg) + c * _deg(d_ref[...])
        o_ref[...] = jnp.maximum(
            jnp.dot(h, w_ref[...], preferred_element_type=jnp.float32)
            + b_ref[...], 0.0)

    ins = [z] + list(parts)
    specs = _row_specs([K]) + [_part_spec(p.shape[2]) for p in parts]
    if stats is not None:
        ins += [degp, stats, g, be]
        specs += [_part_spec(16), _full((8, K)), _full((1, K)), _full((1, K))]
    ins += [Wa, ba]
    specs += [_full((K, D)), _full((1, D))]
    return pl.pallas_call(
        body,
        grid=(GRID,),
        in_specs=specs,
        out_specs=pl.BlockSpec((BN_ROWS, D), lambda i: (i, 0)),
        out_shape=jax.ShapeDtypeStruct((N, D), jnp.float32),
    )(*ins)


def _mm_relu_stats(u, W, b):
    K, D = W.shape

    def body(u_ref, w_ref, b_ref, o_ref, s_ref):
        y = jnp.maximum(
            jnp.dot(u_ref[...], w_ref[...], preferred_element_type=jnp.float32)
            + b_ref[...], 0.0)
        o_ref[...] = y
        _stats_update(s_ref, y, pl.program_id(0))

    return pl.pallas_call(
        body,
        grid=(GRID,),
        in_specs=_row_specs([K]) + [_full((K, D)), _full((1, D))],
        out_specs=[pl.BlockSpec((BN_ROWS, D), lambda i: (i, 0)),
                   _full((8, D))],
        out_shape=[jax.ShapeDtypeStruct((N, D), jnp.float32),
                   jax.ShapeDtypeStruct((8, D), jnp.float32)],
    )(u, W, b)


def _final(ph, pz, stats_h, g03, be03, stats_z, g23, be23,
           Wf1, bf1, Wf2, bf2, Wf3, bf3):
    def body(ph_ref, pz_ref, sh_ref, gh_ref, bh_ref, sz_ref, gz_ref,
             bz_ref, w1_ref, b1_ref, w2_ref, b2_ref, w3_ref, b3_ref, o_ref):
        cnt = pz_ref[0, 0:G, 127:128] + pz_ref[1, 0:G, 127:128]
        ah, ch = _bn_coefs(sh_ref[...], gh_ref[...], bh_ref[...])
        az, cz = _bn_coefs(sz_ref[...], gz_ref[...], bz_ref[...])
        hp = ah * (ph_ref[0, 0:G, :] + ph_ref[1, 0:G, :]) + ch * cnt
        zp = az * (pz_ref[0, 0:G, :] + pz_ref[1, 0:G, :]) + cz * cnt
        cr = jnp.concatenate([hp, zp[:, 0:64]], axis=-1)
        cr = jnp.maximum(
            jnp.dot(cr, w1_ref[...], preferred_element_type=jnp.float32)
            + b1_ref[...], 0.0)
        cr = jnp.maximum(
            jnp.dot(cr, w2_ref[...], preferred_element_type=jnp.float32)
            + b2_ref[...], 0.0)
        cr = jnp.dot(cr, w3_ref[...], preferred_element_type=jnp.float32) \
            + b3_ref[...]
        o_ref[...] = jnp.maximum(cr, 0.0)

    return pl.pallas_call(
        body,
        out_shape=jax.ShapeDtypeStruct((G, 1), jnp.float32),
    )(ph, pz, stats_h, g03, be03, stats_z, g23, be23,
      Wf1, bf1, Wf2, bf2, Wf3, bf3)


def kernel(x, edge_index, batch, W1, b1, W2, b2, W3, b3, g01, be01, g02, be02,
           g03, be03, Wg1a, bg1a, Wg1b, bg1b, g21, be21, Wg2a, bg2a, Wg2b,
           bg2b, g22, be22, Wg3a, bg3a, Wg3b, bg3b, g23, be23, Wf1, bf1, Wf2,
           bf2, Wf3, bf3):
    E = edge_index.shape[1]
    E_PAD = 163840
    src = edge_index[0]
    dst = edge_index[1]
    src_p = jnp.concatenate(
        [src, jnp.zeros((E_PAD - E,), jnp.int32)]).reshape(NW, -1, 128)
    dst_p = jnp.concatenate(
        [dst, jnp.full((E_PAD - E,), N, jnp.int32)]).reshape(NW, -1, 128)
    NSRC_PAD = 10240
    psrc = jnp.concatenate(
        [jnp.arange(N, dtype=jnp.int32),
         jnp.zeros((NSRC_PAD - N,), jnp.int32)]).reshape(NW, -1, 80)
    pdst = jnp.concatenate(
        [batch.astype(jnp.int32),
         jnp.full((NSRC_PAD - N,), G, jnp.int32)]).reshape(NW, -1, 80)

    zNP128 = jnp.zeros((NP, 128), jnp.float32)
    zGP128 = jnp.zeros((GP, 128), jnp.float32)

    sc_edge128 = _make_scatter(128, NP, E_PAD, 128)
    sc_pool128 = _make_scatter(128, GP, NSRC_PAD, 80)

    def edge_scatter(feat):
        return [sc_edge128(feat[:, j:j + 128], src_p, dst_p, zNP128)
                for j in range(0, feat.shape[1], 128)]

    xp = jnp.pad(x, ((0, 0), (0, 11)))
    xp = xp.at[:, 383].set(1.0)
    W1p = jnp.pad(W1, ((0, 11), (0, 0)))
    Wg1ap = jnp.pad(Wg1a, ((0, 11), (0, 0)))
    Wg3bp = jnp.pad(Wg3b, ((0, 0), (0, 64)))
    bg3bp = jnp.pad(bg3b, (0, 64)).at[127].set(1.0)
    g23p = jnp.pad(g23, (0, 64))
    be23p = jnp.pad(be23, (0, 64))
    r1 = lambda v: v.reshape(1, -1)

    px = edge_scatter(xp)
    degp = lax.slice(px[2], (0, 0, 112), (NC, NP, 128))

    hs1 = _mm_dinv(xp, W1p, degp)
    y1, st1 = _gcn_fin(hs1, edge_scatter(hs1), degp, r1(b1))
    hs2 = _mm_dinv(y1, W2, degp, st1, r1(g01), r1(be01))
    y2, st2 = _gcn_fin(hs2, edge_scatter(hs2), degp, r1(b2))
    hs3 = _mm_dinv(y2, W3, degp, st2, r1(g02), r1(be02))
    y3, st3 = _gcn_fin(hs3, edge_scatter(hs3), degp, r1(b3))
    ph = sc_pool128(y3, psrc, pdst, zGP128)

    u1 = _gin_a(xp, px, Wg1ap, r1(bg1a))
    z1, sz1 = _mm_relu_stats(u1, Wg1b, r1(bg1b))
    u2 = _gin_a(z1, edge_scatter(z1), Wg2a, r1(bg2a),
                degp, sz1, r1(g21), r1(be21))
    z2, sz2 = _mm_relu_stats(u2, Wg2b, r1(bg2b))
    u3 = _gin_a(z2, edge_scatter(z2), Wg3a, r1(bg3a),
                degp, sz2, r1(g22), r1(be22))
    z3, sz3 = _mm_relu_stats(u3, Wg3bp, r1(bg3bp))
    pz = sc_pool128(z3, psrc, pdst, zGP128)

    out = _final(ph, pz, st3, r1(g03), r1(be03), sz3, r1(g23p),
                 r1(be23p), Wf1, r1(bf1), Wf2, r1(bf2), Wf3, r1(bf3))
    return out.reshape(-1)
